# baseline (speedup 1.0000x reference)
import functools

import jax
import jax.numpy as jnp
from jax import lax
from jax.experimental import pallas as pl
from jax.experimental.pallas import tpu as pltpu
from jax.experimental.pallas import tpu_sc as plsc

_NUM_WORKERS = 32
_CH = 128


def _retile_table(tableT, tail_rows):
    d, v = tableT.shape
    n_full = v // _CH
    n_bands = d // 8
    slots = -(-n_full // _NUM_WORKERS)
    slots += slots % 2
    n_tail = v - n_full * _CH

    mesh = plsc.VectorSubcoreMesh(core_axis_name="c", subcore_axis_name="s")

    @functools.partial(
        pl.kernel,
        mesh=mesh,
        compiler_params=pltpu.CompilerParams(use_tc_tiling_on_sc=True,
                                             needs_layout_passes=False),
        out_type=jax.ShapeDtypeStruct((v // 16, 8, 128), jnp.float32),
        scratch_types=[
            pltpu.VMEM((n_bands, 8, _CH), jnp.float32),
            pltpu.VMEM((n_bands, 8, _CH), jnp.float32),
            pltpu.VMEM((8, 8, _CH), jnp.float32),
            pltpu.VMEM((8, 8, _CH), jnp.float32),
            pltpu.VMEM((max(n_tail, 1), d), jnp.float32),
            pltpu.SemaphoreType.DMA,
            pltpu.SemaphoreType.DMA,
            pltpu.SemaphoreType.DMA,
            pltpu.SemaphoreType.DMA,
        ],
    )
    def retile_kernel(tabt_hbm, tail_hbm, x_hbm, vb0, vb1, xb0, xb1,
                      tail_v, r0, r1, w0, w1):
        vbuf = (vb0, vb1)
        xbuf = (xb0, xb1)
        rsem = (r0, r1)
        wsem = (w0, w1)

        wid = lax.axis_index("s") * 2 + lax.axis_index("c")

        iota = lax.iota(jnp.int32, 16)
        dd = lax.rem(iota, 8)
        jb = [lax.div(iota, 8) + 2 * k for k in range(4)]

        def e0_of(slot):
            c = jnp.minimum(wid * slots + slot, n_full - 1)
            return pl.multiple_of(c * _CH, _CH)

        def band_reads(slot, b):
            e0 = e0_of(slot)
            return [
                pltpu.make_async_copy(
                    tabt_hbm.at[pl.ds(8 * band, 8), pl.ds(e0, _CH)],
                    vbuf[b].at[band],
                    rsem[b])
                for band in range(n_bands)
            ]

        def x_write_desc(slot, b):
            g0 = lax.div(e0_of(slot), 16)
            return pltpu.make_async_copy(xbuf[b],
                                         x_hbm.at[pl.ds(g0, 8)], wsem[b])

        def transpose_chunk(b):
            @plsc.parallel_loop(0, 8, 1, unroll=2)
            def _(t):
                base = jnp.zeros((16,), jnp.int32) + 16 * t
                for j in range(8):
                    for h in range(2):
                        ee = base + (2 * j + h)
                        for k in range(4):
                            vv = plsc.load_gather(vbuf[b], [jb[k], dd, ee])
                            xbuf[b][t, j, pl.ds(64 * h + 16 * k, 16)] = vv

        for b in range(2):
            for c in band_reads(b, b):
                c.start()
        for b in range(2):
            for c in band_reads(b, b):
                c.wait()
            transpose_chunk(b)
            x_write_desc(b, b).start()
            if 2 + b < slots:
                for c in band_reads(2 + b, b):
                    c.start()

        def outer(s, carry):
            for b in range(2):
                slot = s * 2 + b
                for c in band_reads(slot, b):
                    c.wait()
                x_write_desc(slot - 2, b).wait()
                transpose_chunk(b)
                x_write_desc(slot, b).start()

                @pl.when(slot + 2 < slots)
                def _():
                    for c in band_reads(slot + 2, b):
                        c.start()
            return carry

        lax.fori_loop(1, slots // 2, outer, 0)
        for b in range(2):
            x_write_desc(slots - 2 + b, b).wait()

        if n_tail:
            @pl.when(wid == 0)
            def _():
                pltpu.sync_copy(tail_hbm, tail_v)
                for i in range(n_tail):
                    r = n_full * _CH + i
                    pltpu.sync_copy(
                        tail_v.at[i],
                        x_hbm.at[r // 16, (r % 16) // 2, pl.ds(64 * (r % 2), 64)])

    return retile_kernel(tableT, tail_rows)


def kernel(input, table):
    flat = input.reshape(-1).astype(jnp.int32)
    b_total = flat.shape[0]
    v, d = table.shape
    bpw = b_total // _NUM_WORKERS
    n_chunks = bpw // _CH
    n_bands = d // 8
    tcols = b_total // _CH

    mesh = plsc.VectorSubcoreMesh(core_axis_name="c", subcore_axis_name="s")

    @functools.partial(
        pl.kernel,
        mesh=mesh,
        compiler_params=pltpu.CompilerParams(use_tc_tiling_on_sc=False,
                                             needs_layout_passes=False),
        out_type=jax.ShapeDtypeStruct((n_bands, tcols, 8, _CH), jnp.float32),
        scratch_types=[
            pltpu.VMEM((bpw,), jnp.int32),
            pltpu.VMEM((_CH, d), jnp.float32),
            pltpu.VMEM((_CH, d), jnp.float32),
            pltpu.VMEM((d, _CH), jnp.float32),
            pltpu.VMEM((d, _CH), jnp.float32),
            pltpu.SemaphoreType.DMA,
            pltpu.SemaphoreType.DMA,
            pltpu.SemaphoreType.DMA,
            pltpu.SemaphoreType.DMA,
        ],
    )
    def gather_kernel(idx_hbm, table_hbm, outp_hbm, idx_v,
                      rows0, rows1, tb0, tb1, g0, g1, w0, w1):
        rows = (rows0, rows1)
        tbuf = (tb0, tb1)
        gsem = (g0, g1)
        wsem = (w0, w1)

        wid = lax.axis_index("s") * 2 + lax.axis_index("c")
        base = wid * bpw
        tcol0 = wid * n_chunks
        pltpu.sync_copy(idx_hbm.at[pl.ds(base, bpw)], idx_v)

        iota = lax.iota(jnp.int32, 16)
        row_ids = [iota + 16 * k for k in range(_CH // 16)]

        def gather(t, b):
            return pltpu.async_copy(
                table_hbm.at[idx_v.at[pl.ds(t * _CH, _CH)]], rows[b], gsem[b])

        def out_writes(t, b):
            return [
                pltpu.make_async_copy(
                    tbuf[b].at[pl.ds(a * 8, 8)],
                    outp_hbm.at[a, tcol0 + t],
                    wsem[b])
                for a in range(n_bands)
            ]

        def transpose_chunk(b):
            @plsc.parallel_loop(0, _CH // 16, 1, unroll=2)
            def _(k):
                rv = iota + 16 * k
                for dim in range(d):
                    col = jnp.zeros((16,), jnp.int32) + dim
                    vv = plsc.load_gather(rows[b], [rv, col])
                    tbuf[b][dim, pl.ds(16 * k, 16)] = vv

        g_pending = [gather(0, 0), gather(1, 1)]
        for b in range(2):
            g_pending[b].wait()
            transpose_chunk(b)
            for c in out_writes(b, b):
                c.start()
            if 2 + b < n_chunks:
                gather(2 + b, b)

        def outer(s, carry):
            for b in range(2):
                t = s * 2 + b
                pltpu.make_async_copy(
                    table_hbm.at[idx_v.at[pl.ds(t * _CH, _CH)]],
                    rows[b], gsem[b]).wait()
                for c in out_writes(t - 2, b):
                    c.wait()
                transpose_chunk(b)
                for c in out_writes(t, b):
                    c.start()

                @pl.when(t + 2 < n_chunks)
                def _():
                    gather(t + 2, b)
            return carry

        lax.fori_loop(1, n_chunks // 2, outer, 0)

        for b in range(2):
            for c in out_writes(n_chunks - 2 + b, b):
                c.wait()

    tbl2 = jnp.pad(table, ((0, 0), (0, d))).reshape(2 * v, d)
    outp = gather_kernel(flat * 2, tbl2)
    return outp.transpose(1, 3, 0, 2).reshape(b_total, d)


# speedup vs baseline: 1.2467x; 1.2467x over previous
import functools

import jax
import jax.numpy as jnp
from jax import lax
from jax.experimental import pallas as pl
from jax.experimental.pallas import tpu as pltpu
from jax.experimental.pallas import tpu_sc as plsc

_NUM_WORKERS = 32


def _chunk_size(bpw: int) -> int:
    for ch in range(min(bpw, 400), 0, -8):
        if bpw % ch == 0:
            return ch
    return bpw


@functools.partial(jax.jit, static_argnames=())
def kernel(input, table):
    flat = input.reshape(-1).astype(jnp.int32)
    b_total = flat.shape[0]
    d = table.shape[1]
    bpw = b_total // _NUM_WORKERS
    ch = _chunk_size(bpw)
    n_chunks = bpw // ch
    nbuf = min(4, n_chunks)

    mesh = plsc.VectorSubcoreMesh(core_axis_name="c", subcore_axis_name="s")

    @functools.partial(
        pl.kernel,
        mesh=mesh,
        compiler_params=pltpu.CompilerParams(use_tc_tiling_on_sc=False),
        out_type=jax.ShapeDtypeStruct((b_total, d), jnp.float32),
        scratch_types=(
            [pltpu.VMEM((bpw,), jnp.int32)]
            + [pltpu.VMEM((ch, d), jnp.float32) for _ in range(nbuf)]
            + [pltpu.SemaphoreType.DMA for _ in range(2 * nbuf)]
        ),
    )
    def gather_kernel(idx_hbm, table_hbm, out_hbm, idx_v, *bufs_and_sems):
        rows = bufs_and_sems[:nbuf]
        gsem = bufs_and_sems[nbuf:2 * nbuf]
        wsem = bufs_and_sems[2 * nbuf:3 * nbuf]

        wid = lax.axis_index("s") * 2 + lax.axis_index("c")
        base = wid * bpw
        pltpu.sync_copy(idx_hbm.at[pl.ds(base, bpw)], idx_v)

        def gather(c, b):
            return pltpu.async_copy(
                table_hbm.at[idx_v.at[pl.ds(c * ch, ch)]], rows[b], gsem[b])

        def write(c, b):
            return pltpu.async_copy(
                rows[b], out_hbm.at[pl.ds(base + c * ch, ch)], wsem[b])

        g = [gather(k, k) for k in range(nbuf)]
        w = [None] * nbuf
        for c in range(n_chunks):
            b = c % nbuf
            g[b].wait()
            w[b] = write(c, b)
            nc = c + nbuf
            if nc < n_chunks:
                w[b].wait()
                g[b] = gather(nc, b)
        for k in range(max(0, n_chunks - nbuf), n_chunks):
            w[k % nbuf].wait()

    tbl2 = jnp.pad(table, ((0, 0), (0, d))).reshape(2 * table.shape[0], d)
    return gather_kernel(flat * 2, tbl2)


# speedup vs baseline: 1.8408x; 1.4765x over previous
import functools

import jax
import jax.numpy as jnp
from jax import lax
from jax.experimental import pallas as pl
from jax.experimental.pallas import tpu as pltpu
from jax.experimental.pallas import tpu_sc as plsc

_NUM_WORKERS = 32


def _chunk_size(bpw: int) -> int:
    for ch in range(min(bpw, 400), 0, -8):
        if bpw % ch == 0:
            return ch
    return bpw


@functools.partial(jax.jit, static_argnames=())
def kernel(input, table):
    flat = input.reshape(-1).astype(jnp.int32)
    b_total = flat.shape[0]
    d = table.shape[1]
    bpw = b_total // _NUM_WORKERS
    ch = 128
    n_chunks = bpw // ch
    nbuf = min(4, n_chunks)

    mesh = plsc.VectorSubcoreMesh(core_axis_name="c", subcore_axis_name="s")

    @functools.partial(
        pl.kernel,
        mesh=mesh,
        compiler_params=pltpu.CompilerParams(use_tc_tiling_on_sc=False),
        out_type=jax.ShapeDtypeStruct((b_total // 128, 128, 2 * d), jnp.float32),
        scratch_types=(
            [pltpu.VMEM((bpw,), jnp.int32)]
            + [pltpu.VMEM((ch, d), jnp.float32) for _ in range(nbuf)]
            + [pltpu.SemaphoreType.DMA for _ in range(2 * nbuf)]
        ),
    )
    def gather_kernel(idx_hbm, table_hbm, out_hbm, idx_v, *bufs_and_sems):
        rows = bufs_and_sems[:nbuf]
        gsem = bufs_and_sems[nbuf:2 * nbuf]
        wsem = bufs_and_sems[2 * nbuf:3 * nbuf]

        wid = lax.axis_index("s") * 2 + lax.axis_index("c")
        base = wid * bpw
        pltpu.sync_copy(idx_hbm.at[pl.ds(base, bpw)], idx_v)

        def gather(c, b):
            return pltpu.async_copy(
                table_hbm.at[idx_v.at[pl.ds(c * ch, ch)]], rows[b], gsem[b])

        def write(c, b):
            return pltpu.async_copy(
                rows[b],
                out_hbm.at[(base + c * ch) // 128, :, pl.ds(0, d)], wsem[b])

        g = [gather(k, k) for k in range(nbuf)]
        w = [None] * nbuf
        for c in range(n_chunks):
            b = c % nbuf
            g[b].wait()
            w[b] = write(c, b)
            nc = c + nbuf
            if nc < n_chunks:
                w[b].wait()
                g[b] = gather(nc, b)
        for k in range(max(0, n_chunks - nbuf), n_chunks):
            w[k % nbuf].wait()

    tbl2 = jnp.pad(table, ((0, 0), (0, d))).reshape(2 * table.shape[0], d)
    outp = gather_kernel(flat * 2, tbl2)
    return outp[:, :, 0:d].reshape(b_total, d)


# speedup vs baseline: 1.8550x; 1.0078x over previous
import functools

import jax
import jax.numpy as jnp
from jax import lax
from jax.experimental import pallas as pl
from jax.experimental.pallas import tpu as pltpu
from jax.experimental.pallas import tpu_sc as plsc

_NUM_WORKERS = 32


def _chunk_size(bpw: int) -> int:
    for ch in range(min(bpw, 400), 0, -8):
        if bpw % ch == 0:
            return ch
    return bpw


@functools.partial(jax.jit, static_argnames=())
def kernel(input, table):
    flat = input.reshape(-1).astype(jnp.int32)
    b_total = flat.shape[0]
    d = table.shape[1]
    bpw = b_total // _NUM_WORKERS
    ch = 128
    n_chunks = bpw // ch
    nbuf = min(8, n_chunks)

    mesh = plsc.VectorSubcoreMesh(core_axis_name="c", subcore_axis_name="s")

    @functools.partial(
        pl.kernel,
        mesh=mesh,
        compiler_params=pltpu.CompilerParams(use_tc_tiling_on_sc=False),
        out_type=jax.ShapeDtypeStruct((b_total // 128, 128, 2 * d), jnp.float32),
        scratch_types=(
            [pltpu.VMEM((bpw,), jnp.int32)]
            + [pltpu.VMEM((ch, d), jnp.float32) for _ in range(nbuf)]
            + [pltpu.SemaphoreType.DMA for _ in range(2 * nbuf)]
        ),
    )
    def gather_kernel(idx_hbm, table_hbm, out_hbm, idx_v, *bufs_and_sems):
        rows = bufs_and_sems[:nbuf]
        gsem = bufs_and_sems[nbuf:2 * nbuf]
        wsem = bufs_and_sems[2 * nbuf:3 * nbuf]

        wid = lax.axis_index("s") * 2 + lax.axis_index("c")
        base = wid * bpw
        pltpu.sync_copy(idx_hbm.at[pl.ds(base, bpw)], idx_v)

        def gather(c, b):
            return pltpu.async_copy(
                table_hbm.at[idx_v.at[pl.ds(c * ch, ch)]], rows[b], gsem[b])

        def write(c, b):
            return pltpu.async_copy(
                rows[b],
                out_hbm.at[(base + c * ch) // 128, :, pl.ds(0, d)], wsem[b])

        g = [gather(k, k) for k in range(nbuf)]
        w = [None] * nbuf
        for c in range(n_chunks):
            b = c % nbuf
            g[b].wait()
            w[b] = write(c, b)
            nc = c + nbuf
            if nc < n_chunks:
                w[b].wait()
                g[b] = gather(nc, b)
        for k in range(max(0, n_chunks - nbuf), n_chunks):
            w[k % nbuf].wait()

    tbl2 = jnp.pad(table, ((0, 0), (0, d))).reshape(2 * table.shape[0], d)
    outp = gather_kernel(flat * 2, tbl2)
    return outp[:, :, 0:d].reshape(b_total, d)
